# baseline (device time: 28563 ns/iter reference)
import jax
import jax.numpy as jnp
from jax import lax
from jax.experimental import pallas as pl
from jax.experimental.pallas import tpu as pltpu

N_DEV = 4
BM = 512


def kernel(x, dy, gamma):
    m, d = x.shape
    n_blocks = m // BM

    def body(x_ref, dy_ref, out_ref, comm_ref, send_sems, recv_sems):
        step = pl.program_id(0)

        xv = x_ref[...]
        dyv = dy_ref[...]
        mu = jnp.mean(xv, axis=1, keepdims=True)
        diff = xv - mu
        var = jnp.mean(diff * diff, axis=1, keepdims=True)
        rstd = lax.rsqrt(var + 1e-5)
        xhat = diff * rstd
        dgamma = jnp.sum(dyv * xhat, axis=0)
        dbeta = jnp.sum(dyv, axis=0)

        @pl.when(step == 0)
        def _():
            comm_ref[0, 0, :] = dgamma
            comm_ref[0, 1, :] = dbeta

        @pl.when(step > 0)
        def _():
            comm_ref[0, 0, :] += dgamma
            comm_ref[0, 1, :] += dbeta

        @pl.when(step == n_blocks - 1)
        def _():
            my = lax.axis_index("i")

            barrier_sem = pltpu.get_barrier_semaphore()
            for o in (1, 2, 3):
                pl.semaphore_signal(
                    barrier_sem, inc=1,
                    device_id=((my + o) % N_DEV,),
                    device_id_type=pl.DeviceIdType.MESH,
                )
            pl.semaphore_wait(barrier_sem, 3)

            rdmas = []
            for o in (1, 2, 3):
                rdma = pltpu.make_async_remote_copy(
                    src_ref=comm_ref.at[0],
                    dst_ref=comm_ref.at[N_DEV - o],
                    send_sem=send_sems.at[o - 1],
                    recv_sem=recv_sems.at[N_DEV - o],
                    device_id=((my + o) % N_DEV,),
                    device_id_type=pl.DeviceIdType.MESH,
                )
                rdma.start()
                rdmas.append(rdma)
            for rdma in rdmas:
                rdma.wait()

            out_ref[...] = (
                comm_ref[0] + comm_ref[1] + comm_ref[2] + comm_ref[3]
            )

    return pl.pallas_call(
        body,
        grid=(n_blocks,),
        in_specs=[
            pl.BlockSpec((BM, d), lambda i: (i, 0)),
            pl.BlockSpec((BM, d), lambda i: (i, 0)),
        ],
        out_specs=pl.BlockSpec((2, d), lambda i: (0, 0)),
        out_shape=jax.ShapeDtypeStruct((2, d), jnp.float32),
        scratch_shapes=[
            pltpu.VMEM((N_DEV, 2, d), jnp.float32),
            pltpu.SemaphoreType.DMA((3,)),
            pltpu.SemaphoreType.DMA((N_DEV,)),
        ],
        compiler_params=pltpu.CompilerParams(
            dimension_semantics=("arbitrary",),
            collective_id=0,
        ),
    )(x, dy)
